# baseline (speedup 1.0000x reference)
import functools

import jax
import jax.numpy as jnp
from jax import lax
from jax.experimental import pallas as pl
from jax.experimental.pallas import tpu as pltpu
from jax.experimental.pallas import tpu_sc as plsc

T_STEPS = 4
NUM_CORES = 2
NUM_SUBCORES = 16
NW = NUM_CORES * NUM_SUBCORES
CHUNK = 128
BLK = 1000
SPAD = 512


def _trans_body(h_ref, A_ref, out_ref):
    h = h_ref[...]
    for t in range(out_ref.shape[0]):
        out_ref[t] = jnp.dot(h, A_ref[t], preferred_element_type=jnp.float32)


def _gru_math(h, m, Wz_ref, Uz_ref, bz_ref, Wr_ref, Ur_ref, br_ref,
              Wh_ref, Uh_ref, bh_ref):
    dot = lambda a, b: jnp.dot(a, b, preferred_element_type=jnp.float32)
    z = jax.nn.sigmoid(dot(m, Wz_ref[...]) + dot(h, Uz_ref[...]) + bz_ref[...])
    r = jax.nn.sigmoid(dot(m, Wr_ref[...]) + dot(h, Ur_ref[...]) + br_ref[...])
    h_t = jnp.tanh(dot(m, Wh_ref[...]) + dot(r * h, Uh_ref[...]) + bh_ref[...])
    return (1.0 - z) * h + z * h_t


def _gru_trans_body(h_ref, m2_ref, A_ref, Wz_ref, Uz_ref, bz_ref,
                    Wr_ref, Ur_ref, br_ref, Wh_ref, Uh_ref, bh_ref,
                    hout_ref, trans_ref):
    h = h_ref[...]
    m = m2_ref[0] + m2_ref[1]
    hn = _gru_math(h, m, Wz_ref, Uz_ref, bz_ref, Wr_ref, Ur_ref, br_ref,
                   Wh_ref, Uh_ref, bh_ref)
    hout_ref[...] = hn
    for t in range(trans_ref.shape[0]):
        trans_ref[t] = jnp.dot(hn, A_ref[t], preferred_element_type=jnp.float32)


def _gru_readout_body(h_ref, m2_ref, Wz_ref, Uz_ref, bz_ref,
                      Wr_ref, Ur_ref, br_ref, Wh_ref, Uh_ref, bh_ref,
                      Wg_ref, bg_ref, Wp_ref, bp_ref, starts_ref, out_ref):
    h = h_ref[...]
    m = m2_ref[0] + m2_ref[1]
    hn = _gru_math(h, m, Wz_ref, Uz_ref, bz_ref, Wr_ref, Ur_ref, br_ref,
                   Wh_ref, Uh_ref, bh_ref)
    dot = lambda a, b: jnp.dot(a, b, preferred_element_type=jnp.float32)
    gate = jax.nn.sigmoid(dot(hn, Wg_ref[...]) + bg_ref[0, 0])
    proj = dot(hn, Wp_ref[...]) + bp_ref[0, 0]
    gated = gate * proj
    i = pl.program_id(0)
    blk = h_ref.shape[0]
    rows = (i * blk
            + lax.broadcasted_iota(jnp.int32, (blk, 1), 0)).astype(jnp.float32)
    cnt = jnp.sum((rows >= starts_ref[...]).astype(jnp.float32),
                  axis=1, keepdims=True)
    seg = cnt - 1.0
    g = out_ref.shape[0]
    gidx = lax.broadcasted_iota(jnp.int32, (1, g), 1).astype(jnp.float32)
    onehot = (seg == gidx).astype(jnp.float32)
    contrib = jnp.sum(onehot * gated, axis=0).reshape(g, 1)

    @pl.when(i == 0)
    def _init():
        out_ref[...] = contrib

    @pl.when(i > 0)
    def _acc():
        out_ref[...] += contrib


def _trans0(h, A):
    n, d = h.shape
    nt = A.shape[0]
    return pl.pallas_call(
        _trans_body,
        grid=(n // BLK,),
        in_specs=[
            pl.BlockSpec((BLK, d), lambda i: (i, 0)),
            pl.BlockSpec((nt, d, d), lambda i: (0, 0, 0)),
        ],
        out_specs=pl.BlockSpec((nt, BLK, d), lambda i: (0, i, 0)),
        out_shape=jax.ShapeDtypeStruct((nt, n, d), jnp.float32),
    )(h, A)


def _gru_trans(h, m_parts, A, Wz, Uz, bz2, Wr, Ur, br2, Wh, Uh, bh2):
    n, d = h.shape
    nt = A.shape[0]
    wspec = pl.BlockSpec((d, d), lambda i: (0, 0))
    bspec = pl.BlockSpec((1, d), lambda i: (0, 0))
    return pl.pallas_call(
        _gru_trans_body,
        grid=(n // BLK,),
        in_specs=[
            pl.BlockSpec((BLK, d), lambda i: (i, 0)),
            pl.BlockSpec((2, BLK, d), lambda i: (0, i, 0)),
            pl.BlockSpec((nt, d, d), lambda i: (0, 0, 0)),
            wspec, wspec, bspec, wspec, wspec, bspec, wspec, wspec, bspec,
        ],
        out_specs=[
            pl.BlockSpec((BLK, d), lambda i: (i, 0)),
            pl.BlockSpec((nt, BLK, d), lambda i: (0, i, 0)),
        ],
        out_shape=[
            jax.ShapeDtypeStruct((n, d), jnp.float32),
            jax.ShapeDtypeStruct((nt, n, d), jnp.float32),
        ],
    )(h, m_parts, A, Wz, Uz, bz2, Wr, Ur, br2, Wh, Uh, bh2)


def _gru_readout(h, m_parts, Wz, Uz, bz2, Wr, Ur, br2, Wh, Uh, bh2,
                 Wg, bg2, Wp, bp2, starts_f, g):
    n, d = h.shape
    wspec = pl.BlockSpec((d, d), lambda i: (0, 0))
    bspec = pl.BlockSpec((1, d), lambda i: (0, 0))
    vspec = pl.BlockSpec((d, 1), lambda i: (0, 0))
    sspec = pl.BlockSpec((1, 1), lambda i: (0, 0))
    return pl.pallas_call(
        _gru_readout_body,
        grid=(n // BLK,),
        in_specs=[
            pl.BlockSpec((BLK, d), lambda i: (i, 0)),
            pl.BlockSpec((2, BLK, d), lambda i: (0, i, 0)),
            wspec, wspec, bspec, wspec, wspec, bspec, wspec, wspec, bspec,
            vspec, sspec, vspec, sspec,
            pl.BlockSpec((1, SPAD), lambda i: (0, 0)),
        ],
        out_specs=pl.BlockSpec((g, 1), lambda i: (0, 0)),
        out_shape=jax.ShapeDtypeStruct((g, 1), jnp.float32),
    )(h, m_parts, Wz, Uz, bz2, Wr, Ur, br2, Wh, Uh, bh2,
      Wg, bg2, Wp, bp2, starts_f)


@functools.lru_cache(maxsize=None)
def _make_sc_msg(n_pad, e_pad, d):
    ew = e_pad // NW
    cw = ew // CHUNK
    rows_per_tile = n_pad // NUM_SUBCORES
    nfull = rows_per_tile // CHUNK
    rem = rows_per_tile % CHUNK
    mesh = plsc.VectorSubcoreMesh(core_axis_name="c", subcore_axis_name="s")

    @functools.partial(
        pl.kernel,
        mesh=mesh,
        out_type=jax.ShapeDtypeStruct((NUM_CORES, n_pad, d), jnp.float32),
        scratch_types=[
            [pltpu.VMEM((CHUNK,), jnp.int32) for _ in range(2)],
            [pltpu.VMEM((CHUNK,), jnp.int32) for _ in range(2)],
            pltpu.VMEM((CHUNK, d), jnp.float32),
            pltpu.VMEM_SHARED((n_pad, d), jnp.float32),
            pltpu.SemaphoreType.DMA,
            [pltpu.SemaphoreType.DMA for _ in range(2)],
        ],
    )
    def sc_msg(comb_hbm, dst_hbm, trans_hbm, out_hbm,
               cidx, didx, rows, acc_sh, gsem, isem):
        cid = lax.axis_index("c")
        sid = lax.axis_index("s")
        wid = cid * NUM_SUBCORES + sid
        ebase = wid * ew

        def idx_copy(j, b):
            off = ebase + j * CHUNK
            pltpu.async_copy(comb_hbm.at[pl.ds(off, CHUNK)], cidx[b], isem[b])
            pltpu.async_copy(dst_hbm.at[pl.ds(off, CHUNK)], didx[b], isem[b])

        def idx_wait(b):
            pltpu.make_async_copy(comb_hbm.at[pl.ds(0, CHUNK)], cidx[b],
                                  isem[b]).wait()
            pltpu.make_async_copy(dst_hbm.at[pl.ds(0, CHUNK)], didx[b],
                                  isem[b]).wait()

        idx_copy(0, 0)
        lanes = d // 16

        def zbody(j, carry):
            row = j // lanes
            col = j % lanes
            rows[row, pl.ds(col * 16, 16)] = jnp.zeros((16,), jnp.float32)
            return carry

        lax.fori_loop(0, CHUNK * lanes, zbody, 0)
        base_r = sid * rows_per_tile

        def zdma(k, carry):
            pltpu.sync_copy(rows, acc_sh.at[pl.ds(base_r + k * CHUNK, CHUNK)])
            return carry

        lax.fori_loop(0, nfull, zdma, 0)
        if rem:
            pltpu.sync_copy(rows.at[pl.ds(0, rem)],
                            acc_sh.at[pl.ds(base_r + nfull * CHUNK, rem)])
        plsc.subcore_barrier()

        def chunk(j, b, prefetch):
            idx_wait(b)
            if prefetch:
                idx_copy(j + 1, 1 - b)
            pltpu.async_copy(trans_hbm.at[cidx[b]], rows, gsem).wait()
            pltpu.sync_copy(rows, acc_sh.at[didx[b]], add=True)

        def body(jj, carry):
            chunk(jj * 2, 0, True)
            chunk(jj * 2 + 1, 1, True)
            return carry

        lax.fori_loop(0, cw // 2 - 1, body, 0)
        chunk(cw - 2, 0, True)
        chunk(cw - 1, 1, False)

        plsc.subcore_barrier()
        pltpu.sync_copy(acc_sh.at[pl.ds(base_r, rows_per_tile)],
                        out_hbm.at[cid, pl.ds(base_r, rows_per_tile)])

    return sc_msg


def kernel(node_features, edge_index, edge_type, node_grp_start_with_end,
           A, Wz, Uz, bz, Wr, Ur, br, Wh, Uh, bh, Wp, bp, Wg, bg):
    n, d = node_features.shape
    nt = A.shape[0]
    e = edge_index.shape[1]
    g = node_grp_start_with_end.shape[0] - 1

    n_pad = 128 * ((n + 1 + 127) // 128)
    grain = NW * CHUNK * 8
    e_pad = grain * ((e + grain - 1) // grain)

    src = edge_index[0].astype(jnp.int32)
    dst = edge_index[1].astype(jnp.int32)
    comb = edge_type.astype(jnp.int32) * n + src
    pad_e = e_pad - e
    comb_p = jnp.concatenate([comb, jnp.zeros((pad_e,), jnp.int32)])
    dst_p = jnp.concatenate([dst, jnp.full((pad_e,), n, jnp.int32)])

    bz2, br2, bh2 = bz.reshape(1, d), br.reshape(1, d), bh.reshape(1, d)
    bg2, bp2 = bg.reshape(1, 1), bp.reshape(1, 1)
    starts_f = jnp.full((1, SPAD), 2.0 * n, jnp.float32)
    starts_f = starts_f.at[0, : g + 1].set(
        node_grp_start_with_end.astype(jnp.float32))

    sc_msg = _make_sc_msg(n_pad, e_pad, d)

    h = node_features
    trans = _trans0(h, A).reshape(nt * n, d)
    out = None
    for s in range(T_STEPS):
        m_parts = sc_msg(comb_p, dst_p, trans)
        if s < T_STEPS - 1:
            h, trans4 = _gru_trans(h, m_parts, A, Wz, Uz, bz2,
                                   Wr, Ur, br2, Wh, Uh, bh2)
            trans = trans4.reshape(nt * n, d)
        else:
            out = _gru_readout(h, m_parts, Wz, Uz, bz2, Wr, Ur, br2,
                               Wh, Uh, bh2, Wg, bg2, Wp, bp2, starts_f, g)
    return out


# speedup vs baseline: 1.2397x; 1.2397x over previous
import functools

import jax
import jax.numpy as jnp
from jax import lax
from jax.experimental import pallas as pl
from jax.experimental.pallas import tpu as pltpu
from jax.experimental.pallas import tpu_sc as plsc

T_STEPS = 4
NUM_CORES = 2
NUM_SUBCORES = 16
NW = NUM_CORES * NUM_SUBCORES
CHUNK = 128
BLK = 1000
SPAD = 512


def _trans_body(h_ref, A_ref, out_ref):
    h = h_ref[...]
    for t in range(out_ref.shape[0]):
        out_ref[t] = jnp.dot(h, A_ref[t], preferred_element_type=jnp.float32)


def _gru_math(h, m, Wz_ref, Uz_ref, bz_ref, Wr_ref, Ur_ref, br_ref,
              Wh_ref, Uh_ref, bh_ref):
    dot = lambda a, b: jnp.dot(a, b, preferred_element_type=jnp.float32)
    z = jax.nn.sigmoid(dot(m, Wz_ref[...]) + dot(h, Uz_ref[...]) + bz_ref[...])
    r = jax.nn.sigmoid(dot(m, Wr_ref[...]) + dot(h, Ur_ref[...]) + br_ref[...])
    h_t = jnp.tanh(dot(m, Wh_ref[...]) + dot(r * h, Uh_ref[...]) + bh_ref[...])
    return (1.0 - z) * h + z * h_t


def _gru_trans_body(h_ref, m2_ref, A_ref, Wz_ref, Uz_ref, bz_ref,
                    Wr_ref, Ur_ref, br_ref, Wh_ref, Uh_ref, bh_ref,
                    hout_ref, trans_ref):
    h = h_ref[...]
    m = m2_ref[0] + m2_ref[1]
    hn = _gru_math(h, m, Wz_ref, Uz_ref, bz_ref, Wr_ref, Ur_ref, br_ref,
                   Wh_ref, Uh_ref, bh_ref)
    hout_ref[...] = hn
    for t in range(trans_ref.shape[0]):
        trans_ref[t] = jnp.dot(hn, A_ref[t], preferred_element_type=jnp.float32)


def _gru_readout_body(h_ref, m2_ref, Wz_ref, Uz_ref, bz_ref,
                      Wr_ref, Ur_ref, br_ref, Wh_ref, Uh_ref, bh_ref,
                      Wg_ref, bg_ref, Wp_ref, bp_ref, starts_ref, out_ref):
    h = h_ref[...]
    m = m2_ref[0] + m2_ref[1]
    hn = _gru_math(h, m, Wz_ref, Uz_ref, bz_ref, Wr_ref, Ur_ref, br_ref,
                   Wh_ref, Uh_ref, bh_ref)
    dot = lambda a, b: jnp.dot(a, b, preferred_element_type=jnp.float32)
    gate = jax.nn.sigmoid(dot(hn, Wg_ref[...]) + bg_ref[0, 0])
    proj = dot(hn, Wp_ref[...]) + bp_ref[0, 0]
    gated = gate * proj
    i = pl.program_id(0)
    blk = h_ref.shape[0]
    rows = (i * blk
            + lax.broadcasted_iota(jnp.int32, (blk, 1), 0)).astype(jnp.float32)
    cnt = jnp.sum((rows >= starts_ref[...]).astype(jnp.float32),
                  axis=1, keepdims=True)
    seg = cnt - 1.0
    g = out_ref.shape[0]
    gidx = lax.broadcasted_iota(jnp.int32, (1, g), 1).astype(jnp.float32)
    onehot = (seg == gidx).astype(jnp.float32)
    contrib = jnp.sum(onehot * gated, axis=0).reshape(g, 1)

    @pl.when(i == 0)
    def _init():
        out_ref[...] = contrib

    @pl.when(i > 0)
    def _acc():
        out_ref[...] += contrib


def _trans0(h, A):
    n, d = h.shape
    nt = A.shape[0]
    return pl.pallas_call(
        _trans_body,
        grid=(n // BLK,),
        in_specs=[
            pl.BlockSpec((BLK, d), lambda i: (i, 0)),
            pl.BlockSpec((nt, d, d), lambda i: (0, 0, 0)),
        ],
        out_specs=pl.BlockSpec((nt, BLK, d), lambda i: (0, i, 0)),
        out_shape=jax.ShapeDtypeStruct((nt, n, d), jnp.float32),
    )(h, A)


def _gru_trans(h, m_parts, A, Wz, Uz, bz2, Wr, Ur, br2, Wh, Uh, bh2):
    n, d = h.shape
    nt = A.shape[0]
    wspec = pl.BlockSpec((d, d), lambda i: (0, 0))
    bspec = pl.BlockSpec((1, d), lambda i: (0, 0))
    return pl.pallas_call(
        _gru_trans_body,
        grid=(n // BLK,),
        in_specs=[
            pl.BlockSpec((BLK, d), lambda i: (i, 0)),
            pl.BlockSpec((2, BLK, d), lambda i: (0, i, 0)),
            pl.BlockSpec((nt, d, d), lambda i: (0, 0, 0)),
            wspec, wspec, bspec, wspec, wspec, bspec, wspec, wspec, bspec,
        ],
        out_specs=[
            pl.BlockSpec((BLK, d), lambda i: (i, 0)),
            pl.BlockSpec((nt, BLK, d), lambda i: (0, i, 0)),
        ],
        out_shape=[
            jax.ShapeDtypeStruct((n, d), jnp.float32),
            jax.ShapeDtypeStruct((nt, n, d), jnp.float32),
        ],
    )(h, m_parts, A, Wz, Uz, bz2, Wr, Ur, br2, Wh, Uh, bh2)


def _gru_readout(h, m_parts, Wz, Uz, bz2, Wr, Ur, br2, Wh, Uh, bh2,
                 Wg, bg2, Wp, bp2, starts_f, g):
    n, d = h.shape
    wspec = pl.BlockSpec((d, d), lambda i: (0, 0))
    bspec = pl.BlockSpec((1, d), lambda i: (0, 0))
    vspec = pl.BlockSpec((d, 1), lambda i: (0, 0))
    sspec = pl.BlockSpec((1, 1), lambda i: (0, 0))
    return pl.pallas_call(
        _gru_readout_body,
        grid=(n // BLK,),
        in_specs=[
            pl.BlockSpec((BLK, d), lambda i: (i, 0)),
            pl.BlockSpec((2, BLK, d), lambda i: (0, i, 0)),
            wspec, wspec, bspec, wspec, wspec, bspec, wspec, wspec, bspec,
            vspec, sspec, vspec, sspec,
            pl.BlockSpec((1, SPAD), lambda i: (0, 0)),
        ],
        out_specs=pl.BlockSpec((g, 1), lambda i: (0, 0)),
        out_shape=jax.ShapeDtypeStruct((g, 1), jnp.float32),
    )(h, m_parts, Wz, Uz, bz2, Wr, Ur, br2, Wh, Uh, bh2,
      Wg, bg2, Wp, bp2, starts_f)


CW0_FRAC = 0.45


@functools.lru_cache(maxsize=None)
def _make_sc_msg(n_pad, e_pad, d):
    chunks_total = e_pad // CHUNK
    per_pair = chunks_total // NUM_SUBCORES
    cw0 = max(1, int(per_pair * CW0_FRAC))
    cw1 = per_pair - cw0
    rows_per_tile = n_pad // NUM_SUBCORES
    nfull = rows_per_tile // CHUNK
    rem = rows_per_tile % CHUNK
    mesh = plsc.VectorSubcoreMesh(core_axis_name="c", subcore_axis_name="s")

    @functools.partial(
        pl.kernel,
        mesh=mesh,
        out_type=jax.ShapeDtypeStruct((NUM_CORES, n_pad, d), jnp.float32),
        scratch_types=[
            pltpu.VMEM((CHUNK,), jnp.int32),
            pltpu.VMEM((CHUNK,), jnp.int32),
            pltpu.VMEM((CHUNK, d), jnp.float32),
            pltpu.VMEM_SHARED((n_pad, d), jnp.float32),
            pltpu.SemaphoreType.DMA,
        ],
    )
    def sc_msg(comb_hbm, dst_hbm, trans_hbm, out_hbm,
               idx_v, dst_v, rows_v, acc_sh, sem):
        cid = lax.axis_index("c")
        sid = lax.axis_index("s")

        lanes = d // 16

        def zbody(j, carry):
            row = j // lanes
            col = j % lanes
            rows_v[row, pl.ds(col * 16, 16)] = jnp.zeros((16,), jnp.float32)
            return carry

        lax.fori_loop(0, CHUNK * lanes, zbody, 0)
        base_r = sid * rows_per_tile

        def zdma(k, carry):
            pltpu.sync_copy(rows_v, acc_sh.at[pl.ds(base_r + k * CHUNK, CHUNK)])
            return carry

        lax.fori_loop(0, nfull, zdma, 0)
        if rem:
            pltpu.sync_copy(rows_v.at[pl.ds(0, rem)],
                            acc_sh.at[pl.ds(base_r + nfull * CHUNK, rem)])
        plsc.subcore_barrier()

        def body_at(ebase):
            def body(j, carry):
                off = ebase + j * CHUNK
                pltpu.sync_copy(comb_hbm.at[pl.ds(off, CHUNK)], idx_v)
                pltpu.sync_copy(dst_hbm.at[pl.ds(off, CHUNK)], dst_v)
                pltpu.async_copy(trans_hbm.at[idx_v], rows_v, sem).wait()
                pltpu.sync_copy(rows_v, acc_sh.at[dst_v], add=True)
                return carry
            return body

        @pl.when(cid == 0)
        def _core0():
            lax.fori_loop(0, cw0, body_at(sid * cw0 * CHUNK), 0)

        @pl.when(cid == 1)
        def _core1():
            lax.fori_loop(0, cw1,
                          body_at((NUM_SUBCORES * cw0 + sid * cw1) * CHUNK), 0)

        plsc.subcore_barrier()
        pltpu.sync_copy(acc_sh.at[pl.ds(base_r, rows_per_tile)],
                        out_hbm.at[cid, pl.ds(base_r, rows_per_tile)])

    return sc_msg


def kernel(node_features, edge_index, edge_type, node_grp_start_with_end,
           A, Wz, Uz, bz, Wr, Ur, br, Wh, Uh, bh, Wp, bp, Wg, bg):
    n, d = node_features.shape
    nt = A.shape[0]
    e = edge_index.shape[1]
    g = node_grp_start_with_end.shape[0] - 1

    n_pad = 128 * ((n + 1 + 127) // 128)
    grain = NW * CHUNK
    e_pad = grain * ((e + grain - 1) // grain)

    src = edge_index[0].astype(jnp.int32)
    dst = edge_index[1].astype(jnp.int32)
    comb = edge_type.astype(jnp.int32) * n + src
    pad_e = e_pad - e
    comb_p = jnp.concatenate([comb, jnp.zeros((pad_e,), jnp.int32)])
    dst_p = jnp.concatenate([dst, jnp.full((pad_e,), n, jnp.int32)])

    bz2, br2, bh2 = bz.reshape(1, d), br.reshape(1, d), bh.reshape(1, d)
    bg2, bp2 = bg.reshape(1, 1), bp.reshape(1, 1)
    starts_f = jnp.full((1, SPAD), 2.0 * n, jnp.float32)
    starts_f = starts_f.at[0, : g + 1].set(
        node_grp_start_with_end.astype(jnp.float32))

    sc_msg = _make_sc_msg(n_pad, e_pad, d)

    h = node_features
    trans = _trans0(h, A).reshape(nt * n, d)
    out = None
    for s in range(T_STEPS):
        m_parts = sc_msg(comb_p, dst_p, trans)
        if s < T_STEPS - 1:
            h, trans4 = _gru_trans(h, m_parts, A, Wz, Uz, bz2,
                                   Wr, Ur, br2, Wh, Uh, bh2)
            trans = trans4.reshape(nt * n, d)
        else:
            out = _gru_readout(h, m_parts, Wz, Uz, bz2, Wr, Ur, br2,
                               Wh, Uh, bh2, Wg, bg2, Wp, bp2, starts_f, g)
    return out


# speedup vs baseline: 1.3460x; 1.0857x over previous
import functools

import jax
import jax.numpy as jnp
from jax import lax
from jax.experimental import pallas as pl
from jax.experimental.pallas import tpu as pltpu
from jax.experimental.pallas import tpu_sc as plsc

T_STEPS = 4
NUM_CORES = 2
NUM_SUBCORES = 16
NW = NUM_CORES * NUM_SUBCORES
CHUNK = 128
BLK = 1000
SPAD = 512


def _trans_body(h_ref, A_ref, out_ref):
    h = h_ref[...]
    for t in range(out_ref.shape[0]):
        out_ref[t] = jnp.dot(h, A_ref[t], preferred_element_type=jnp.float32)


def _gru_math(h, m, Wz_ref, Uz_ref, bz_ref, Wr_ref, Ur_ref, br_ref,
              Wh_ref, Uh_ref, bh_ref):
    dot = lambda a, b: jnp.dot(a, b, preferred_element_type=jnp.float32)
    z = jax.nn.sigmoid(dot(m, Wz_ref[...]) + dot(h, Uz_ref[...]) + bz_ref[...])
    r = jax.nn.sigmoid(dot(m, Wr_ref[...]) + dot(h, Ur_ref[...]) + br_ref[...])
    h_t = jnp.tanh(dot(m, Wh_ref[...]) + dot(r * h, Uh_ref[...]) + bh_ref[...])
    return (1.0 - z) * h + z * h_t


def _gru_trans_body(h_ref, m2_ref, A_ref, Wz_ref, Uz_ref, bz_ref,
                    Wr_ref, Ur_ref, br_ref, Wh_ref, Uh_ref, bh_ref,
                    hout_ref, trans_ref):
    h = h_ref[...]
    m = m2_ref[0] + m2_ref[1]
    hn = _gru_math(h, m, Wz_ref, Uz_ref, bz_ref, Wr_ref, Ur_ref, br_ref,
                   Wh_ref, Uh_ref, bh_ref)
    hout_ref[...] = hn
    for t in range(trans_ref.shape[0]):
        trans_ref[t] = jnp.dot(hn, A_ref[t], preferred_element_type=jnp.float32)


def _gru_readout_body(h_ref, m2_ref, Wz_ref, Uz_ref, bz_ref,
                      Wr_ref, Ur_ref, br_ref, Wh_ref, Uh_ref, bh_ref,
                      Wg_ref, bg_ref, Wp_ref, bp_ref, starts_ref, out_ref):
    h = h_ref[...]
    m = m2_ref[0] + m2_ref[1]
    hn = _gru_math(h, m, Wz_ref, Uz_ref, bz_ref, Wr_ref, Ur_ref, br_ref,
                   Wh_ref, Uh_ref, bh_ref)
    dot = lambda a, b: jnp.dot(a, b, preferred_element_type=jnp.float32)
    gate = jax.nn.sigmoid(dot(hn, Wg_ref[...]) + bg_ref[0, 0])
    proj = dot(hn, Wp_ref[...]) + bp_ref[0, 0]
    gated = gate * proj
    i = pl.program_id(0)
    blk = h_ref.shape[0]
    rows = (i * blk
            + lax.broadcasted_iota(jnp.int32, (blk, 1), 0)).astype(jnp.float32)
    cnt = jnp.sum((rows >= starts_ref[...]).astype(jnp.float32),
                  axis=1, keepdims=True)
    seg = cnt - 1.0
    g = out_ref.shape[0]
    gidx = lax.broadcasted_iota(jnp.int32, (1, g), 1).astype(jnp.float32)
    onehot = (seg == gidx).astype(jnp.float32)
    contrib = jnp.sum(onehot * gated, axis=0).reshape(g, 1)

    @pl.when(i == 0)
    def _init():
        out_ref[...] = contrib

    @pl.when(i > 0)
    def _acc():
        out_ref[...] += contrib


def _trans0(h, A):
    n, d = h.shape
    nt = A.shape[0]
    return pl.pallas_call(
        _trans_body,
        grid=(n // BLK,),
        in_specs=[
            pl.BlockSpec((BLK, d), lambda i: (i, 0)),
            pl.BlockSpec((nt, d, d), lambda i: (0, 0, 0)),
        ],
        out_specs=pl.BlockSpec((nt, BLK, d), lambda i: (0, i, 0)),
        out_shape=jax.ShapeDtypeStruct((nt, n, d), jnp.float32),
    )(h, A)


def _gru_trans(h, m_parts, A, Wz, Uz, bz2, Wr, Ur, br2, Wh, Uh, bh2):
    n, d = h.shape
    nt = A.shape[0]
    wspec = pl.BlockSpec((d, d), lambda i: (0, 0))
    bspec = pl.BlockSpec((1, d), lambda i: (0, 0))
    return pl.pallas_call(
        _gru_trans_body,
        grid=(n // BLK,),
        in_specs=[
            pl.BlockSpec((BLK, d), lambda i: (i, 0)),
            pl.BlockSpec((2, BLK, d), lambda i: (0, i, 0)),
            pl.BlockSpec((nt, d, d), lambda i: (0, 0, 0)),
            wspec, wspec, bspec, wspec, wspec, bspec, wspec, wspec, bspec,
        ],
        out_specs=[
            pl.BlockSpec((BLK, d), lambda i: (i, 0)),
            pl.BlockSpec((nt, BLK, d), lambda i: (0, i, 0)),
        ],
        out_shape=[
            jax.ShapeDtypeStruct((n, d), jnp.float32),
            jax.ShapeDtypeStruct((nt, n, d), jnp.float32),
        ],
    )(h, m_parts, A, Wz, Uz, bz2, Wr, Ur, br2, Wh, Uh, bh2)


def _gru_readout(h, m_parts, Wz, Uz, bz2, Wr, Ur, br2, Wh, Uh, bh2,
                 Wg, bg2, Wp, bp2, starts_f, g):
    n, d = h.shape
    wspec = pl.BlockSpec((d, d), lambda i: (0, 0))
    bspec = pl.BlockSpec((1, d), lambda i: (0, 0))
    vspec = pl.BlockSpec((d, 1), lambda i: (0, 0))
    sspec = pl.BlockSpec((1, 1), lambda i: (0, 0))
    return pl.pallas_call(
        _gru_readout_body,
        grid=(n // BLK,),
        in_specs=[
            pl.BlockSpec((BLK, d), lambda i: (i, 0)),
            pl.BlockSpec((2, BLK, d), lambda i: (0, i, 0)),
            wspec, wspec, bspec, wspec, wspec, bspec, wspec, wspec, bspec,
            vspec, sspec, vspec, sspec,
            pl.BlockSpec((1, SPAD), lambda i: (0, 0)),
        ],
        out_specs=pl.BlockSpec((g, 1), lambda i: (0, 0)),
        out_shape=jax.ShapeDtypeStruct((g, 1), jnp.float32),
    )(h, m_parts, Wz, Uz, bz2, Wr, Ur, br2, Wh, Uh, bh2,
      Wg, bg2, Wp, bp2, starts_f)


CW0_FRAC = 0.55


@functools.lru_cache(maxsize=None)
def _make_sc_msg(n_pad, e_pad, d):
    chunks_total = e_pad // CHUNK
    per_pair = chunks_total // NUM_SUBCORES
    cw0 = max(1, int(per_pair * CW0_FRAC))
    cw1 = per_pair - cw0
    rows_per_tile = n_pad // NUM_SUBCORES
    nfull = rows_per_tile // CHUNK
    rem = rows_per_tile % CHUNK
    mesh = plsc.VectorSubcoreMesh(core_axis_name="c", subcore_axis_name="s")

    @functools.partial(
        pl.kernel,
        mesh=mesh,
        out_type=jax.ShapeDtypeStruct((NUM_CORES, n_pad, d), jnp.float32),
        scratch_types=[
            pltpu.VMEM((CHUNK,), jnp.int32),
            pltpu.VMEM((CHUNK,), jnp.int32),
            pltpu.VMEM((CHUNK, d), jnp.float32),
            pltpu.VMEM_SHARED((n_pad, d), jnp.float32),
            pltpu.SemaphoreType.DMA,
        ],
    )
    def sc_msg(comb_hbm, dst_hbm, trans_hbm, out_hbm,
               idx_v, dst_v, rows_v, acc_sh, sem):
        cid = lax.axis_index("c")
        sid = lax.axis_index("s")

        lanes = d // 16

        def zbody(j, carry):
            row = j // lanes
            col = j % lanes
            rows_v[row, pl.ds(col * 16, 16)] = jnp.zeros((16,), jnp.float32)
            return carry

        lax.fori_loop(0, CHUNK * lanes, zbody, 0)
        base_r = sid * rows_per_tile

        def zdma(k, carry):
            pltpu.sync_copy(rows_v, acc_sh.at[pl.ds(base_r + k * CHUNK, CHUNK)])
            return carry

        lax.fori_loop(0, nfull, zdma, 0)
        if rem:
            pltpu.sync_copy(rows_v.at[pl.ds(0, rem)],
                            acc_sh.at[pl.ds(base_r + nfull * CHUNK, rem)])
        plsc.subcore_barrier()

        def body_at(ebase):
            def body(j, carry):
                off = ebase + j * CHUNK
                pltpu.sync_copy(comb_hbm.at[pl.ds(off, CHUNK)], idx_v)
                pltpu.sync_copy(dst_hbm.at[pl.ds(off, CHUNK)], dst_v)
                pltpu.async_copy(trans_hbm.at[idx_v], rows_v, sem).wait()
                pltpu.sync_copy(rows_v, acc_sh.at[dst_v], add=True)
                return carry
            return body

        @pl.when(cid == 0)
        def _core0():
            lax.fori_loop(0, cw0, body_at(sid * cw0 * CHUNK), 0)

        @pl.when(cid == 1)
        def _core1():
            lax.fori_loop(0, cw1,
                          body_at((NUM_SUBCORES * cw0 + sid * cw1) * CHUNK), 0)

        plsc.subcore_barrier()
        pltpu.sync_copy(acc_sh.at[pl.ds(base_r, rows_per_tile)],
                        out_hbm.at[cid, pl.ds(base_r, rows_per_tile)])

    return sc_msg


def kernel(node_features, edge_index, edge_type, node_grp_start_with_end,
           A, Wz, Uz, bz, Wr, Ur, br, Wh, Uh, bh, Wp, bp, Wg, bg):
    n, d = node_features.shape
    nt = A.shape[0]
    e = edge_index.shape[1]
    g = node_grp_start_with_end.shape[0] - 1

    n_pad = 128 * ((n + 1 + 127) // 128)
    grain = NW * CHUNK
    e_pad = grain * ((e + grain - 1) // grain)

    src = edge_index[0].astype(jnp.int32)
    dst = edge_index[1].astype(jnp.int32)
    comb = edge_type.astype(jnp.int32) * n + src
    pad_e = e_pad - e
    comb_p = jnp.concatenate([comb, jnp.zeros((pad_e,), jnp.int32)])
    dst_p = jnp.concatenate([dst, jnp.full((pad_e,), n, jnp.int32)])

    bz2, br2, bh2 = bz.reshape(1, d), br.reshape(1, d), bh.reshape(1, d)
    bg2, bp2 = bg.reshape(1, 1), bp.reshape(1, 1)
    starts_f = jnp.full((1, SPAD), 2.0 * n, jnp.float32)
    starts_f = starts_f.at[0, : g + 1].set(
        node_grp_start_with_end.astype(jnp.float32))

    sc_msg = _make_sc_msg(n_pad, e_pad, d)

    h = node_features
    trans = _trans0(h, A).reshape(nt * n, d)
    out = None
    for s in range(T_STEPS):
        m_parts = sc_msg(comb_p, dst_p, trans)
        if s < T_STEPS - 1:
            h, trans4 = _gru_trans(h, m_parts, A, Wz, Uz, bz2,
                                   Wr, Ur, br2, Wh, Uh, bh2)
            trans = trans4.reshape(nt * n, d)
        else:
            out = _gru_readout(h, m_parts, Wz, Uz, bz2, Wr, Ur, br2,
                               Wh, Uh, bh2, Wg, bg2, Wp, bp2, starts_f, g)
    return out


# speedup vs baseline: 1.3994x; 1.0397x over previous
import functools

import jax
import jax.numpy as jnp
from jax import lax
from jax.experimental import pallas as pl
from jax.experimental.pallas import tpu as pltpu
from jax.experimental.pallas import tpu_sc as plsc

T_STEPS = 4
NUM_CORES = 2
NUM_SUBCORES = 16
NW = NUM_CORES * NUM_SUBCORES
CHUNK = 128
BLK = 1000
SPAD = 512


def _trans_body(h_ref, A_ref, out_ref):
    h = h_ref[...]
    for t in range(out_ref.shape[0]):
        out_ref[t] = jnp.dot(h, A_ref[t], preferred_element_type=jnp.float32)


def _gru_math(h, m, Wz_ref, Uz_ref, bz_ref, Wr_ref, Ur_ref, br_ref,
              Wh_ref, Uh_ref, bh_ref):
    dot = lambda a, b: jnp.dot(a, b, preferred_element_type=jnp.float32)
    z = jax.nn.sigmoid(dot(m, Wz_ref[...]) + dot(h, Uz_ref[...]) + bz_ref[...])
    r = jax.nn.sigmoid(dot(m, Wr_ref[...]) + dot(h, Ur_ref[...]) + br_ref[...])
    h_t = jnp.tanh(dot(m, Wh_ref[...]) + dot(r * h, Uh_ref[...]) + bh_ref[...])
    return (1.0 - z) * h + z * h_t


def _gru_trans_body(h_ref, m2_ref, A_ref, Wz_ref, Uz_ref, bz_ref,
                    Wr_ref, Ur_ref, br_ref, Wh_ref, Uh_ref, bh_ref,
                    hout_ref, trans_ref):
    h = h_ref[...]
    m = m2_ref[0] + m2_ref[1]
    hn = _gru_math(h, m, Wz_ref, Uz_ref, bz_ref, Wr_ref, Ur_ref, br_ref,
                   Wh_ref, Uh_ref, bh_ref)
    hout_ref[...] = hn
    for t in range(trans_ref.shape[0]):
        trans_ref[t] = jnp.dot(hn, A_ref[t], preferred_element_type=jnp.float32)


def _gru_readout_body(h_ref, m2_ref, Wz_ref, Uz_ref, bz_ref,
                      Wr_ref, Ur_ref, br_ref, Wh_ref, Uh_ref, bh_ref,
                      Wg_ref, bg_ref, Wp_ref, bp_ref, starts_ref, out_ref):
    h = h_ref[...]
    m = m2_ref[0] + m2_ref[1]
    hn = _gru_math(h, m, Wz_ref, Uz_ref, bz_ref, Wr_ref, Ur_ref, br_ref,
                   Wh_ref, Uh_ref, bh_ref)
    dot = lambda a, b: jnp.dot(a, b, preferred_element_type=jnp.float32)
    gate = jax.nn.sigmoid(dot(hn, Wg_ref[...]) + bg_ref[0, 0])
    proj = dot(hn, Wp_ref[...]) + bp_ref[0, 0]
    gated = gate * proj
    i = pl.program_id(0)
    blk = h_ref.shape[0]
    rows = (i * blk
            + lax.broadcasted_iota(jnp.int32, (blk, 1), 0)).astype(jnp.float32)
    cnt = jnp.sum((rows >= starts_ref[...]).astype(jnp.float32),
                  axis=1, keepdims=True)
    seg = cnt - 1.0
    g = out_ref.shape[0]
    gidx = lax.broadcasted_iota(jnp.int32, (1, g), 1).astype(jnp.float32)
    onehot = (seg == gidx).astype(jnp.float32)
    contrib = jnp.sum(onehot * gated, axis=0).reshape(g, 1)

    @pl.when(i == 0)
    def _init():
        out_ref[...] = contrib

    @pl.when(i > 0)
    def _acc():
        out_ref[...] += contrib


def _trans0(h, A):
    n, d = h.shape
    nt = A.shape[0]
    return pl.pallas_call(
        _trans_body,
        grid=(n // BLK,),
        in_specs=[
            pl.BlockSpec((BLK, d), lambda i: (i, 0)),
            pl.BlockSpec((nt, d, d), lambda i: (0, 0, 0)),
        ],
        out_specs=pl.BlockSpec((nt, BLK, d), lambda i: (0, i, 0)),
        out_shape=jax.ShapeDtypeStruct((nt, n, d), jnp.float32),
    )(h, A)


def _gru_trans(h, m_parts, A, Wz, Uz, bz2, Wr, Ur, br2, Wh, Uh, bh2):
    n, d = h.shape
    nt = A.shape[0]
    wspec = pl.BlockSpec((d, d), lambda i: (0, 0))
    bspec = pl.BlockSpec((1, d), lambda i: (0, 0))
    return pl.pallas_call(
        _gru_trans_body,
        grid=(n // BLK,),
        in_specs=[
            pl.BlockSpec((BLK, d), lambda i: (i, 0)),
            pl.BlockSpec((2, BLK, d), lambda i: (0, i, 0)),
            pl.BlockSpec((nt, d, d), lambda i: (0, 0, 0)),
            wspec, wspec, bspec, wspec, wspec, bspec, wspec, wspec, bspec,
        ],
        out_specs=[
            pl.BlockSpec((BLK, d), lambda i: (i, 0)),
            pl.BlockSpec((nt, BLK, d), lambda i: (0, i, 0)),
        ],
        out_shape=[
            jax.ShapeDtypeStruct((n, d), jnp.float32),
            jax.ShapeDtypeStruct((nt, n, d), jnp.float32),
        ],
    )(h, m_parts, A, Wz, Uz, bz2, Wr, Ur, br2, Wh, Uh, bh2)


def _gru_readout(h, m_parts, Wz, Uz, bz2, Wr, Ur, br2, Wh, Uh, bh2,
                 Wg, bg2, Wp, bp2, starts_f, g):
    n, d = h.shape
    wspec = pl.BlockSpec((d, d), lambda i: (0, 0))
    bspec = pl.BlockSpec((1, d), lambda i: (0, 0))
    vspec = pl.BlockSpec((d, 1), lambda i: (0, 0))
    sspec = pl.BlockSpec((1, 1), lambda i: (0, 0))
    return pl.pallas_call(
        _gru_readout_body,
        grid=(n // BLK,),
        in_specs=[
            pl.BlockSpec((BLK, d), lambda i: (i, 0)),
            pl.BlockSpec((2, BLK, d), lambda i: (0, i, 0)),
            wspec, wspec, bspec, wspec, wspec, bspec, wspec, wspec, bspec,
            vspec, sspec, vspec, sspec,
            pl.BlockSpec((1, SPAD), lambda i: (0, 0)),
        ],
        out_specs=pl.BlockSpec((g, 1), lambda i: (0, 0)),
        out_shape=jax.ShapeDtypeStruct((g, 1), jnp.float32),
    )(h, m_parts, Wz, Uz, bz2, Wr, Ur, br2, Wh, Uh, bh2,
      Wg, bg2, Wp, bp2, starts_f)


CW0_FRAC = 0.59


@functools.lru_cache(maxsize=None)
def _make_sc_msg(n_pad, e_pad, d):
    chunks_total = e_pad // CHUNK
    per_pair = chunks_total // NUM_SUBCORES
    cw0 = max(1, int(per_pair * CW0_FRAC))
    cw1 = per_pair - cw0
    rows_per_tile = n_pad // NUM_SUBCORES
    nfull = rows_per_tile // CHUNK
    rem = rows_per_tile % CHUNK
    mesh = plsc.VectorSubcoreMesh(core_axis_name="c", subcore_axis_name="s")

    @functools.partial(
        pl.kernel,
        mesh=mesh,
        out_type=jax.ShapeDtypeStruct((NUM_CORES, n_pad, d), jnp.float32),
        scratch_types=[
            pltpu.VMEM((CHUNK,), jnp.int32),
            pltpu.VMEM((CHUNK,), jnp.int32),
            pltpu.VMEM((CHUNK, d), jnp.float32),
            pltpu.VMEM_SHARED((n_pad, d), jnp.float32),
            pltpu.SemaphoreType.DMA,
        ],
    )
    def sc_msg(comb_hbm, dst_hbm, trans_hbm, out_hbm,
               idx_v, dst_v, rows_v, acc_sh, sem):
        cid = lax.axis_index("c")
        sid = lax.axis_index("s")

        lanes = d // 16

        def zbody(j, carry):
            row = j // lanes
            col = j % lanes
            rows_v[row, pl.ds(col * 16, 16)] = jnp.zeros((16,), jnp.float32)
            return carry

        lax.fori_loop(0, CHUNK * lanes, zbody, 0)
        base_r = sid * rows_per_tile

        def zdma(k, carry):
            pltpu.sync_copy(rows_v, acc_sh.at[pl.ds(base_r + k * CHUNK, CHUNK)])
            return carry

        lax.fori_loop(0, nfull, zdma, 0)
        if rem:
            pltpu.sync_copy(rows_v.at[pl.ds(0, rem)],
                            acc_sh.at[pl.ds(base_r + nfull * CHUNK, rem)])
        plsc.subcore_barrier()

        def body_at(ebase):
            def body(j, carry):
                off = ebase + j * CHUNK
                pltpu.sync_copy(comb_hbm.at[pl.ds(off, CHUNK)], idx_v)
                pltpu.sync_copy(dst_hbm.at[pl.ds(off, CHUNK)], dst_v)
                pltpu.async_copy(trans_hbm.at[idx_v], rows_v, sem).wait()
                pltpu.sync_copy(rows_v, acc_sh.at[dst_v], add=True)
                return carry
            return body

        @pl.when(cid == 0)
        def _core0():
            lax.fori_loop(0, cw0, body_at(sid * cw0 * CHUNK), 0)

        @pl.when(cid == 1)
        def _core1():
            lax.fori_loop(0, cw1,
                          body_at((NUM_SUBCORES * cw0 + sid * cw1) * CHUNK), 0)

        plsc.subcore_barrier()
        pltpu.sync_copy(acc_sh.at[pl.ds(base_r, rows_per_tile)],
                        out_hbm.at[cid, pl.ds(base_r, rows_per_tile)])

    return sc_msg


def kernel(node_features, edge_index, edge_type, node_grp_start_with_end,
           A, Wz, Uz, bz, Wr, Ur, br, Wh, Uh, bh, Wp, bp, Wg, bg):
    n, d = node_features.shape
    nt = A.shape[0]
    e = edge_index.shape[1]
    g = node_grp_start_with_end.shape[0] - 1

    n_pad = 128 * ((n + 1 + 127) // 128)
    grain = NW * CHUNK
    e_pad = grain * ((e + grain - 1) // grain)

    src = edge_index[0].astype(jnp.int32)
    dst = edge_index[1].astype(jnp.int32)
    comb = edge_type.astype(jnp.int32) * n + src
    pad_e = e_pad - e
    comb_p = jnp.concatenate([comb, jnp.zeros((pad_e,), jnp.int32)])
    dst_p = jnp.concatenate([dst, jnp.full((pad_e,), n, jnp.int32)])

    bz2, br2, bh2 = bz.reshape(1, d), br.reshape(1, d), bh.reshape(1, d)
    bg2, bp2 = bg.reshape(1, 1), bp.reshape(1, 1)
    starts_f = jnp.full((1, SPAD), 2.0 * n, jnp.float32)
    starts_f = starts_f.at[0, : g + 1].set(
        node_grp_start_with_end.astype(jnp.float32))

    sc_msg = _make_sc_msg(n_pad, e_pad, d)

    h = node_features
    trans = _trans0(h, A).reshape(nt * n, d)
    out = None
    for s in range(T_STEPS):
        m_parts = sc_msg(comb_p, dst_p, trans)
        if s < T_STEPS - 1:
            h, trans4 = _gru_trans(h, m_parts, A, Wz, Uz, bz2,
                                   Wr, Ur, br2, Wh, Uh, bh2)
            trans = trans4.reshape(nt * n, d)
        else:
            out = _gru_readout(h, m_parts, Wz, Uz, bz2, Wr, Ur, br2,
                               Wh, Uh, bh2, Wg, bg2, Wp, bp2, starts_f, g)
    return out


# speedup vs baseline: 1.4300x; 1.0219x over previous
import functools

import jax
import jax.numpy as jnp
from jax import lax
from jax.experimental import pallas as pl
from jax.experimental.pallas import tpu as pltpu
from jax.experimental.pallas import tpu_sc as plsc

T_STEPS = 4
NUM_CORES = 2
NUM_SUBCORES = 16
NW = NUM_CORES * NUM_SUBCORES
CHUNK = 128
BLK = 1000
SPAD = 512


def _trans_body(h_ref, A_ref, out_ref):
    h = h_ref[...]
    for t in range(out_ref.shape[0]):
        out_ref[t] = jnp.dot(h, A_ref[t], preferred_element_type=jnp.float32)


def _gru_math(h, m, Wz_ref, Uz_ref, bz_ref, Wr_ref, Ur_ref, br_ref,
              Wh_ref, Uh_ref, bh_ref):
    dot = lambda a, b: jnp.dot(a, b, preferred_element_type=jnp.float32)
    z = jax.nn.sigmoid(dot(m, Wz_ref[...]) + dot(h, Uz_ref[...]) + bz_ref[...])
    r = jax.nn.sigmoid(dot(m, Wr_ref[...]) + dot(h, Ur_ref[...]) + br_ref[...])
    h_t = jnp.tanh(dot(m, Wh_ref[...]) + dot(r * h, Uh_ref[...]) + bh_ref[...])
    return (1.0 - z) * h + z * h_t


def _gru_trans_body(h_ref, m2_ref, A_ref, Wz_ref, Uz_ref, bz_ref,
                    Wr_ref, Ur_ref, br_ref, Wh_ref, Uh_ref, bh_ref,
                    hout_ref, trans_ref):
    h = h_ref[...]
    m = m2_ref[0] + m2_ref[1]
    hn = _gru_math(h, m, Wz_ref, Uz_ref, bz_ref, Wr_ref, Ur_ref, br_ref,
                   Wh_ref, Uh_ref, bh_ref)
    hout_ref[...] = hn
    for t in range(trans_ref.shape[0]):
        trans_ref[t] = jnp.dot(hn, A_ref[t], preferred_element_type=jnp.float32)


def _gru_readout_body(h_ref, m2_ref, Wz_ref, Uz_ref, bz_ref,
                      Wr_ref, Ur_ref, br_ref, Wh_ref, Uh_ref, bh_ref,
                      Wg_ref, bg_ref, Wp_ref, bp_ref, starts_ref, out_ref):
    h = h_ref[...]
    m = m2_ref[0] + m2_ref[1]
    hn = _gru_math(h, m, Wz_ref, Uz_ref, bz_ref, Wr_ref, Ur_ref, br_ref,
                   Wh_ref, Uh_ref, bh_ref)
    dot = lambda a, b: jnp.dot(a, b, preferred_element_type=jnp.float32)
    gate = jax.nn.sigmoid(dot(hn, Wg_ref[...]) + bg_ref[0, 0])
    proj = dot(hn, Wp_ref[...]) + bp_ref[0, 0]
    gated = gate * proj
    i = pl.program_id(0)
    blk = h_ref.shape[0]
    rows = (i * blk
            + lax.broadcasted_iota(jnp.int32, (blk, 1), 0)).astype(jnp.float32)
    cnt = jnp.sum((rows >= starts_ref[...]).astype(jnp.float32),
                  axis=1, keepdims=True)
    seg = cnt - 1.0
    g = out_ref.shape[0]
    gidx = lax.broadcasted_iota(jnp.int32, (1, g), 1).astype(jnp.float32)
    onehot = (seg == gidx).astype(jnp.float32)
    contrib = jnp.sum(onehot * gated, axis=0).reshape(g, 1)

    @pl.when(i == 0)
    def _init():
        out_ref[...] = contrib

    @pl.when(i > 0)
    def _acc():
        out_ref[...] += contrib


def _trans0(h, A):
    n, d = h.shape
    nt = A.shape[0]
    return pl.pallas_call(
        _trans_body,
        grid=(n // BLK,),
        in_specs=[
            pl.BlockSpec((BLK, d), lambda i: (i, 0)),
            pl.BlockSpec((nt, d, d), lambda i: (0, 0, 0)),
        ],
        out_specs=pl.BlockSpec((nt, BLK, d), lambda i: (0, i, 0)),
        out_shape=jax.ShapeDtypeStruct((nt, n, d), jnp.float32),
    )(h, A)


def _gru_trans(h, m_parts, A, Wz, Uz, bz2, Wr, Ur, br2, Wh, Uh, bh2):
    n, d = h.shape
    nt = A.shape[0]
    wspec = pl.BlockSpec((d, d), lambda i: (0, 0))
    bspec = pl.BlockSpec((1, d), lambda i: (0, 0))
    return pl.pallas_call(
        _gru_trans_body,
        grid=(n // BLK,),
        in_specs=[
            pl.BlockSpec((BLK, d), lambda i: (i, 0)),
            pl.BlockSpec((2, BLK, d), lambda i: (0, i, 0)),
            pl.BlockSpec((nt, d, d), lambda i: (0, 0, 0)),
            wspec, wspec, bspec, wspec, wspec, bspec, wspec, wspec, bspec,
        ],
        out_specs=[
            pl.BlockSpec((BLK, d), lambda i: (i, 0)),
            pl.BlockSpec((nt, BLK, d), lambda i: (0, i, 0)),
        ],
        out_shape=[
            jax.ShapeDtypeStruct((n, d), jnp.float32),
            jax.ShapeDtypeStruct((nt, n, d), jnp.float32),
        ],
    )(h, m_parts, A, Wz, Uz, bz2, Wr, Ur, br2, Wh, Uh, bh2)


def _gru_readout(h, m_parts, Wz, Uz, bz2, Wr, Ur, br2, Wh, Uh, bh2,
                 Wg, bg2, Wp, bp2, starts_f, g):
    n, d = h.shape
    wspec = pl.BlockSpec((d, d), lambda i: (0, 0))
    bspec = pl.BlockSpec((1, d), lambda i: (0, 0))
    vspec = pl.BlockSpec((d, 1), lambda i: (0, 0))
    sspec = pl.BlockSpec((1, 1), lambda i: (0, 0))
    return pl.pallas_call(
        _gru_readout_body,
        grid=(n // BLK,),
        in_specs=[
            pl.BlockSpec((BLK, d), lambda i: (i, 0)),
            pl.BlockSpec((2, BLK, d), lambda i: (0, i, 0)),
            wspec, wspec, bspec, wspec, wspec, bspec, wspec, wspec, bspec,
            vspec, sspec, vspec, sspec,
            pl.BlockSpec((1, SPAD), lambda i: (0, 0)),
        ],
        out_specs=pl.BlockSpec((g, 1), lambda i: (0, 0)),
        out_shape=jax.ShapeDtypeStruct((g, 1), jnp.float32),
    )(h, m_parts, Wz, Uz, bz2, Wr, Ur, br2, Wh, Uh, bh2,
      Wg, bg2, Wp, bp2, starts_f)


CW0_FRAC = 0.62


@functools.lru_cache(maxsize=None)
def _make_sc_msg(n_pad, e_pad, d):
    chunks_total = e_pad // CHUNK
    per_pair = chunks_total // NUM_SUBCORES
    cw0 = max(1, int(per_pair * CW0_FRAC))
    cw1 = per_pair - cw0
    rows_per_tile = n_pad // NUM_SUBCORES
    nfull = rows_per_tile // CHUNK
    rem = rows_per_tile % CHUNK
    mesh = plsc.VectorSubcoreMesh(core_axis_name="c", subcore_axis_name="s")

    @functools.partial(
        pl.kernel,
        mesh=mesh,
        out_type=jax.ShapeDtypeStruct((NUM_CORES, n_pad, d), jnp.float32),
        scratch_types=[
            pltpu.VMEM((CHUNK,), jnp.int32),
            pltpu.VMEM((CHUNK,), jnp.int32),
            pltpu.VMEM((CHUNK, d), jnp.float32),
            pltpu.VMEM_SHARED((n_pad, d), jnp.float32),
            pltpu.SemaphoreType.DMA,
        ],
    )
    def sc_msg(comb_hbm, dst_hbm, trans_hbm, out_hbm,
               idx_v, dst_v, rows_v, acc_sh, sem):
        cid = lax.axis_index("c")
        sid = lax.axis_index("s")

        lanes = d // 16

        def zbody(j, carry):
            row = j // lanes
            col = j % lanes
            rows_v[row, pl.ds(col * 16, 16)] = jnp.zeros((16,), jnp.float32)
            return carry

        lax.fori_loop(0, CHUNK * lanes, zbody, 0)
        base_r = sid * rows_per_tile

        def zdma(k, carry):
            pltpu.sync_copy(rows_v, acc_sh.at[pl.ds(base_r + k * CHUNK, CHUNK)])
            return carry

        lax.fori_loop(0, nfull, zdma, 0)
        if rem:
            pltpu.sync_copy(rows_v.at[pl.ds(0, rem)],
                            acc_sh.at[pl.ds(base_r + nfull * CHUNK, rem)])
        plsc.subcore_barrier()

        def body_at(ebase):
            def body(j, carry):
                off = ebase + j * CHUNK
                pltpu.sync_copy(comb_hbm.at[pl.ds(off, CHUNK)], idx_v)
                pltpu.sync_copy(dst_hbm.at[pl.ds(off, CHUNK)], dst_v)
                pltpu.async_copy(trans_hbm.at[idx_v], rows_v, sem).wait()
                pltpu.sync_copy(rows_v, acc_sh.at[dst_v], add=True)
                return carry
            return body

        @pl.when(cid == 0)
        def _core0():
            lax.fori_loop(0, cw0, body_at(sid * cw0 * CHUNK), 0)

        @pl.when(cid == 1)
        def _core1():
            lax.fori_loop(0, cw1,
                          body_at((NUM_SUBCORES * cw0 + sid * cw1) * CHUNK), 0)

        plsc.subcore_barrier()
        pltpu.sync_copy(acc_sh.at[pl.ds(base_r, rows_per_tile)],
                        out_hbm.at[cid, pl.ds(base_r, rows_per_tile)])

    return sc_msg


def kernel(node_features, edge_index, edge_type, node_grp_start_with_end,
           A, Wz, Uz, bz, Wr, Ur, br, Wh, Uh, bh, Wp, bp, Wg, bg):
    n, d = node_features.shape
    nt = A.shape[0]
    e = edge_index.shape[1]
    g = node_grp_start_with_end.shape[0] - 1

    n_pad = 128 * ((n + 1 + 127) // 128)
    grain = NW * CHUNK
    e_pad = grain * ((e + grain - 1) // grain)

    src = edge_index[0].astype(jnp.int32)
    dst = edge_index[1].astype(jnp.int32)
    comb = edge_type.astype(jnp.int32) * n + src
    pad_e = e_pad - e
    comb_p = jnp.concatenate([comb, jnp.zeros((pad_e,), jnp.int32)])
    dst_p = jnp.concatenate([dst, jnp.full((pad_e,), n, jnp.int32)])

    bz2, br2, bh2 = bz.reshape(1, d), br.reshape(1, d), bh.reshape(1, d)
    bg2, bp2 = bg.reshape(1, 1), bp.reshape(1, 1)
    starts_f = jnp.full((1, SPAD), 2.0 * n, jnp.float32)
    starts_f = starts_f.at[0, : g + 1].set(
        node_grp_start_with_end.astype(jnp.float32))

    sc_msg = _make_sc_msg(n_pad, e_pad, d)

    h = node_features
    trans = _trans0(h, A).reshape(nt * n, d)
    out = None
    for s in range(T_STEPS):
        m_parts = sc_msg(comb_p, dst_p, trans)
        if s < T_STEPS - 1:
            h, trans4 = _gru_trans(h, m_parts, A, Wz, Uz, bz2,
                                   Wr, Ur, br2, Wh, Uh, bh2)
            trans = trans4.reshape(nt * n, d)
        else:
            out = _gru_readout(h, m_parts, Wz, Uz, bz2, Wr, Ur, br2,
                               Wh, Uh, bh2, Wg, bg2, Wp, bp2, starts_f, g)
    return out
